# Initial kernel scaffold; baseline (speedup 1.0000x reference)
import jax
import jax.numpy as jnp
from jax.experimental import pallas as pl


def kernel(x, MLP_output, W_enc, b_enc, decoder, b_dec, W_skip):
    raise NotImplementedError("write your pallas kernel here")


# speedup vs baseline: 6.8237x; 6.8237x over previous
import functools

import jax
import jax.numpy as jnp
from jax import lax
from jax.experimental import pallas as pl
from jax.experimental.pallas import tpu as pltpu

_BR = 128
_BL = 2048
_K_VALUES = (32, 128, 256)


def _k1_body(x_ref, w_ref, b_ref, out_ref):
    acc = lax.dot_general(x_ref[...], w_ref[...],
                          (((1,), (1,)), ((), ())),
                          preferred_element_type=jnp.float32)
    out_ref[...] = acc + b_ref[...]


def _orderable_int(p):
    b = lax.bitcast_convert_type(p, jnp.int32)
    mask = b >> 31
    return b ^ (mask & jnp.int32(0x7FFFFFFF))


def _kth_largest(s, k):
    c0 = jnp.sum((s >= 0).astype(jnp.int32), axis=1, keepdims=True)
    ge0 = c0 >= k
    lo = jnp.where(ge0, jnp.int32(0), jnp.int32(-2147483648))
    hi = jnp.where(ge0, jnp.int32(2147483647), jnp.int32(-1))

    def body(_, carry):
        lo, hi = carry
        d = hi - lo
        mid = lo + (d >> 1) + (d & 1)
        cnt = jnp.sum((s >= mid).astype(jnp.int32), axis=1, keepdims=True)
        ge = cnt >= k
        return jnp.where(ge, mid, lo), jnp.where(ge, hi, mid - 1)

    lo, hi = lax.fori_loop(0, 31, body, (lo, hi))
    return lo


def _int_to_float(sv):
    b = jnp.where(sv >= 0, sv, sv ^ jnp.int32(0x7FFFFFFF))
    return lax.bitcast_convert_type(b, jnp.float32)


def _k2_body(pre_ref, thr_ref):
    s = _orderable_int(pre_ref[...])
    br = s.shape[0]
    t = [_int_to_float(_kth_largest(s, k)) for k in _K_VALUES]
    npos = jnp.sum((s >= 1).astype(jnp.int32), axis=1,
                   keepdims=True).astype(jnp.float32)
    cols = lax.broadcasted_iota(jnp.int32, (br, 128), 1)
    out = jnp.where(cols == 0, t[0], 0.0)
    out = out + jnp.where(cols == 1, t[1], 0.0)
    out = out + jnp.where(cols == 2, t[2], 0.0)
    out = out + jnp.where(cols == 3, npos, 0.0)
    thr_ref[...] = out


def _k3_body(pre_ref, thr_ref, dec_ref, xb_ref, x32_ref, m_ref, wskip_ref,
             bdec_ref, me_ref, md_ref, scal_ref, cs_ref,
             accA, accB, accC, accS, *, n_l):
    i = pl.program_id(0)
    l = pl.program_id(1)

    @pl.when(jnp.logical_and(i == 0, l == 0))
    def _():
        scal_ref[...] = jnp.zeros_like(scal_ref)
        cs_ref[...] = jnp.zeros_like(cs_ref)

    p = pre_ref[...]
    t32 = thr_ref[:, 0:1]
    t128 = thr_ref[:, 1:2]
    t256 = thr_ref[:, 2:3]
    npos = thr_ref[:, 3:4]

    r = jnp.maximum(p, 0.0)
    zero = jnp.zeros_like(p)
    g1 = jnp.where(p >= t32, r, zero)
    g2 = jnp.where(p >= t128, r, zero)
    g3 = jnp.where(p >= t256, r, zero)
    me_ref[...] = (g1 + g2 + g3) * jnp.float32(1.0 / 3.0)

    dec = dec_ref[...]
    eA = g1.astype(jnp.bfloat16)
    eB = (g2 - g1).astype(jnp.bfloat16)
    eC = (g3 - g2).astype(jnp.bfloat16)

    @pl.when(l == 0)
    def _():
        skip = jnp.dot(xb_ref[...], wskip_ref[...],
                       preferred_element_type=jnp.float32) + bdec_ref[...]
        accS[...] = skip
        x32 = x32_ref[...]
        cs_ref[...] += jnp.sum(x32, axis=0, keepdims=True)
        sumx2 = jnp.sum(x32 * x32)
        cols = lax.broadcasted_iota(jnp.int32, scal_ref.shape, 1)
        scal_ref[...] += jnp.where(cols == 4, sumx2, 0.0)

    dA = jnp.dot(eA, dec, preferred_element_type=jnp.float32)
    dB = jnp.dot(eB, dec, preferred_element_type=jnp.float32)
    dC = jnp.dot(eC, dec, preferred_element_type=jnp.float32)

    @pl.when(l == 0)
    def _():
        accA[...] = dA
        accB[...] = dB
        accC[...] = dC

    @pl.when(l > 0)
    def _():
        accA[...] += dA
        accB[...] += dB
        accC[...] += dC

    @pl.when(l == n_l - 1)
    def _():
        A = accA[...]
        AB = A + accB[...]
        ABC = AB + accC[...]
        s = accS[...]
        md_ref[...] = s + (A + AB + ABC) * jnp.float32(1.0 / 3.0)
        resid = m_ref[...] - s
        l1 = jnp.sum((resid - A) ** 2)
        l2 = jnp.sum((resid - AB) ** 2)
        l3 = jnp.sum((resid - ABC) ** 2)
        l0p = jnp.sum(jnp.minimum(npos, jnp.float32(256.0)))
        cols = lax.broadcasted_iota(jnp.int32, scal_ref.shape, 1)
        upd = jnp.where(cols == 0, l1, 0.0)
        upd = upd + jnp.where(cols == 1, l2, 0.0)
        upd = upd + jnp.where(cols == 2, l3, 0.0)
        upd = upd + jnp.where(cols == 3, l0p, 0.0)
        scal_ref[...] += upd


def kernel(x, MLP_output, W_enc, b_enc, decoder, b_dec, W_skip):
    B, D = x.shape
    L = W_enc.shape[0]
    br = _BR if B % _BR == 0 else B
    bl = _BL if L % _BL == 0 else L
    n_r, n_l = B // br, L // bl

    xb = x.astype(jnp.bfloat16)
    decb = decoder.astype(jnp.bfloat16)
    wskipb = W_skip.astype(jnp.bfloat16)
    benc2 = b_enc.reshape(1, L)
    bdec2 = b_dec.reshape(1, D)

    pre = pl.pallas_call(
        _k1_body,
        grid=(n_r, n_l),
        in_specs=[
            pl.BlockSpec((br, D), lambda i, l: (i, 0)),
            pl.BlockSpec((bl, D), lambda i, l: (l, 0)),
            pl.BlockSpec((1, bl), lambda i, l: (0, l)),
        ],
        out_specs=pl.BlockSpec((br, bl), lambda i, l: (i, l)),
        out_shape=jax.ShapeDtypeStruct((B, L), jnp.float32),
    )(x, W_enc, benc2)

    thr = pl.pallas_call(
        _k2_body,
        grid=(n_r,),
        in_specs=[pl.BlockSpec((br, L), lambda i: (i, 0))],
        out_specs=pl.BlockSpec((br, 128), lambda i: (i, 0)),
        out_shape=jax.ShapeDtypeStruct((B, 128), jnp.float32),
    )(pre)

    mean_enc, mean_dec, scal, colsum = pl.pallas_call(
        functools.partial(_k3_body, n_l=n_l),
        grid=(n_r, n_l),
        in_specs=[
            pl.BlockSpec((br, bl), lambda i, l: (i, l)),
            pl.BlockSpec((br, 128), lambda i, l: (i, 0)),
            pl.BlockSpec((bl, D), lambda i, l: (l, 0)),
            pl.BlockSpec((br, D), lambda i, l: (i, 0)),
            pl.BlockSpec((br, D), lambda i, l: (i, 0)),
            pl.BlockSpec((br, D), lambda i, l: (i, 0)),
            pl.BlockSpec((D, D), lambda i, l: (0, 0)),
            pl.BlockSpec((1, D), lambda i, l: (0, 0)),
        ],
        out_specs=[
            pl.BlockSpec((br, bl), lambda i, l: (i, l)),
            pl.BlockSpec((br, D), lambda i, l: (i, 0)),
            pl.BlockSpec((1, 128), lambda i, l: (0, 0)),
            pl.BlockSpec((1, D), lambda i, l: (0, 0)),
        ],
        out_shape=[
            jax.ShapeDtypeStruct((B, L), jnp.float32),
            jax.ShapeDtypeStruct((B, D), jnp.float32),
            jax.ShapeDtypeStruct((1, 128), jnp.float32),
            jax.ShapeDtypeStruct((1, D), jnp.float32),
        ],
        scratch_shapes=[
            pltpu.VMEM((br, D), jnp.float32),
            pltpu.VMEM((br, D), jnp.float32),
            pltpu.VMEM((br, D), jnp.float32),
            pltpu.VMEM((br, D), jnp.float32),
        ],
    )(pre, thr, decb, xb, x, MLP_output, wskipb, bdec2)

    total_variance = scal[0, 4] - jnp.sum(colsum[0] ** 2) / jnp.float32(B)
    total_loss = (scal[0, 0] + scal[0, 1] + scal[0, 2]) / total_variance
    l0 = scal[0, 3].astype(jnp.int32)
    return (mean_enc, mean_dec, total_loss, l0)
